# Initial kernel scaffold; baseline (speedup 1.0000x reference)
import jax
import jax.numpy as jnp
from jax.experimental import pallas as pl


def kernel(val, index):
    raise NotImplementedError("write your pallas kernel here")


# speedup vs baseline: 2.0477x; 2.0477x over previous
import functools

import jax
import jax.numpy as jnp
from jax import lax
from jax.experimental import pallas as pl
from jax.experimental.pallas import tpu as pltpu
import jax.experimental.pallas.tpu_sc as plsc

N = 7_077_888
FLAT = 28_311_552
CHUNK = 1 << 20
NCHUNK = 27
ROUNDS = 14
DUMP = 1 << 16
BLK = 2048
ROWS = BLK // 128
TILE_PAIRS = N // 16
NBLK = TILE_PAIRS // BLK
SLICE = CHUNK // 16
ZB = 8192

_mesh = plsc.VectorSubcoreMesh(core_axis_name="c", subcore_axis_name="s")


def _body(val_hbm, idx_hbm, out_hbm, inv, ini, sidx, zbuf, chunkbuf):
    c = lax.axis_index("c")
    s = lax.axis_index("s")

    zv = jnp.zeros((16,), jnp.float32)

    def _zb_init(i, carry):
        zbuf[pl.ds(i * 16, 16)] = zv
        return carry

    lax.fori_loop(0, ZB // 16, _zb_init, 0)

    def _round(r, carry):
        ch = jnp.where(c == 0, r, ROUNDS + r).astype(jnp.int32)
        active = ch < NCHUNK
        base = ch * CHUNK

        @pl.when(active)
        def _zero():
            for k in range(SLICE // ZB):
                pltpu.sync_copy(
                    zbuf, chunkbuf.at[pl.ds(s * SLICE + k * ZB, ZB)])

        plsc.subcore_barrier()

        @pl.when(active)
        def _scan():
            def blk_body(b, carry):
                row0 = s * (TILE_PAIRS // 128) + b * ROWS
                pltpu.sync_copy(idx_hbm.at[pl.ds(row0, ROWS)], ini)
                pltpu.sync_copy(val_hbm.at[pl.ds(row0, ROWS)], inv)
                for j in range(ROWS):
                    irow = ini.at[j]
                    srow = sidx.at[j]
                    for k in range(8):
                        idx = irow[pl.ds(k * 16, 16)]
                        loc = idx - base
                        oob = plsc.bitcast(loc, jnp.uint32) >= jnp.uint32(CHUNK)
                        dmp = CHUNK + (idx & (DUMP - 1))
                        srow[pl.ds(k * 16, 16)] = jnp.where(oob, dmp, loc)
                for j in range(ROWS):
                    pltpu.sync_copy(
                        inv.at[j], chunkbuf.at[sidx.at[j]], add=True)
                return carry

            lax.fori_loop(0, NBLK, blk_body, 0)

        plsc.subcore_barrier()

        @pl.when(active)
        def _copyout():
            pltpu.sync_copy(
                chunkbuf.at[pl.ds(s * SLICE, SLICE)],
                out_hbm.at[pl.ds(base + s * SLICE, SLICE)])

        plsc.subcore_barrier()
        return carry

    lax.fori_loop(0, ROUNDS, _round, 0)


@jax.jit
def _scatter_add(val2d, idx2d):
    return pl.kernel(
        _body,
        out_type=jax.ShapeDtypeStruct((FLAT,), jnp.float32),
        mesh=_mesh,
        scratch_types=[
            pltpu.VMEM((ROWS, 128), jnp.float32),
            pltpu.VMEM((ROWS, 128), jnp.int32),
            pltpu.VMEM((ROWS, 128), jnp.int32),
            pltpu.VMEM((ZB,), jnp.float32),
            pltpu.VMEM_SHARED((CHUNK + DUMP,), jnp.float32),
        ],
    )(val2d, idx2d)


def kernel(val, index):
    B, H, W, C = index.shape
    val2d = val.reshape(N // 128, 128)
    idx2d = index.astype(jnp.int32).reshape(N // 128, 128)
    flat = _scatter_add(val2d, idx2d)
    return flat.reshape(B, H * 2, W * 2, C)


# speedup vs baseline: 3.4690x; 1.6941x over previous
import jax
import jax.numpy as jnp
from jax import lax
from jax.experimental import pallas as pl
from jax.experimental.pallas import tpu as pltpu
import jax.experimental.pallas.tpu_sc as plsc

N = 7_077_888
FLAT = 28_311_552
NCHUNK = 16
CHUNK = FLAT // NCHUNK
ROUNDS = NCHUNK // 2
DUMP = 1 << 16
BLK = 2048
ROWS = BLK // 128
TILE_PAIRS = N // 16
NBLK = TILE_PAIRS // BLK
SLICE = CHUNK // 16
ZB = 8192

_mesh = plsc.VectorSubcoreMesh(core_axis_name="c", subcore_axis_name="s")


def _body(val_hbm, idx_hbm, out_hbm, inv, ini, sidx, zbuf, chunkbuf):
    c = lax.axis_index("c")
    s = lax.axis_index("s")

    zv = jnp.zeros((16,), jnp.float32)

    def _zb_init(i, carry):
        zbuf[pl.ds(i * 16, 16)] = zv
        return carry

    lax.fori_loop(0, ZB // 16, _zb_init, 0)

    def _round(r, carry):
        ch = (c * ROUNDS + r).astype(jnp.int32)
        base = ch * CHUNK

        for k in range(13):
            pltpu.sync_copy(zbuf, chunkbuf.at[pl.ds(s * SLICE + k * ZB, ZB)])
        pltpu.sync_copy(
            zbuf.at[pl.ds(0, ZB // 2)],
            chunkbuf.at[pl.ds(s * SLICE + 13 * ZB, ZB // 2)])

        plsc.subcore_barrier()

        def blk_body(b, carry2):
            row0 = s * (TILE_PAIRS // 128) + b * ROWS
            pltpu.sync_copy(idx_hbm.at[pl.ds(row0, ROWS)], ini)
            pltpu.sync_copy(val_hbm.at[pl.ds(row0, ROWS)], inv)
            for j in range(ROWS):
                irow = ini.at[j]
                srow = sidx.at[j]
                for k in range(8):
                    idx = irow[pl.ds(k * 16, 16)]
                    loc = idx - base
                    oob = plsc.bitcast(loc, jnp.uint32) >= jnp.uint32(CHUNK)
                    dmp = CHUNK + (idx & (DUMP - 1))
                    srow[pl.ds(k * 16, 16)] = jnp.where(oob, dmp, loc)
            for j in range(ROWS):
                pltpu.sync_copy(
                    inv.at[j], chunkbuf.at[sidx.at[j]], add=True)
            return carry2

        lax.fori_loop(0, NBLK, blk_body, 0)

        plsc.subcore_barrier()

        pltpu.sync_copy(
            chunkbuf.at[pl.ds(s * SLICE, SLICE)],
            out_hbm.at[pl.ds(base + s * SLICE, SLICE)])

        plsc.subcore_barrier()
        return carry

    lax.fori_loop(0, ROUNDS, _round, 0)


@jax.jit
def _scatter_add(val2d, idx2d):
    return pl.kernel(
        _body,
        out_type=jax.ShapeDtypeStruct((FLAT,), jnp.float32),
        mesh=_mesh,
        scratch_types=[
            pltpu.VMEM((ROWS, 128), jnp.float32),
            pltpu.VMEM((ROWS, 128), jnp.int32),
            pltpu.VMEM((ROWS, 128), jnp.int32),
            pltpu.VMEM((ZB,), jnp.float32),
            pltpu.VMEM_SHARED((CHUNK + DUMP,), jnp.float32),
        ],
    )(val2d, idx2d)


def kernel(val, index):
    B, H, W, C = index.shape
    val2d = val.reshape(N // 128, 128)
    idx2d = index.astype(jnp.int32).reshape(N // 128, 128)
    flat = _scatter_add(val2d, idx2d)
    return flat.reshape(B, H * 2, W * 2, C)
